# baseline (speedup 1.0000x reference)
import jax
import jax.numpy as jnp
from jax import lax
from jax.experimental import pallas as pl
from jax.experimental.pallas import tpu as pltpu

_TOPK = 0.1
_LAYER = 2
_K = 51
_KP = 56
_H = 8


def _wsel_body(te_ref, fr_ref, tgt_ref, tsel_ref, wsel_ref):
    T, D = te_ref.shape[1], te_ref.shape[2]
    te = te_ref[0]
    fr = fr_ref[0]
    tgt = tgt_ref[0]

    dn = (((1,), (1,)), ((), ()))
    num_row = lax.dot_general(fr, te, dn,
                              preferred_element_type=jnp.float32)
    ones_d = jnp.ones((1, D), jnp.float32)
    sqn_row = lax.dot_general(ones_d, te * te, dn,
                              preferred_element_type=jnp.float32)
    fr_norm = jnp.sqrt(jnp.sum(fr * fr))
    w_row = num_row / jnp.maximum(jnp.sqrt(sqn_row) * fr_norm, 1e-8)
    w_col = w_row.reshape(T, 1)

    ic = lax.broadcasted_iota(jnp.int32, (T, T), 0)
    ir = lax.broadcasted_iota(jnp.int32, (T, T), 1)
    beats = (w_col > w_row) | ((w_col == w_row) & (ic < ir))
    beats_f = jnp.where(beats, 1.0, 0.0)
    ones_t = jnp.ones((1, T), jnp.float32)
    rank_row = lax.dot_general(ones_t, beats_f, (((1,), (0,)), ((), ())),
                               preferred_element_type=jnp.float32)

    seq_len = jnp.sum((tgt != 0).astype(jnp.float32))
    cc = jnp.minimum(jnp.ceil(seq_len * _TOPK), float(_K))

    j_col = lax.broadcasted_iota(jnp.int32, (_KP, 1), 0).astype(jnp.float32)
    onehot = jnp.where(rank_row == j_col, 1.0, 0.0)
    t_col_f = lax.broadcasted_iota(jnp.int32, (T, 1), 0).astype(jnp.float32)
    cat = jnp.concatenate([t_col_f, w_col], axis=1)
    tw = lax.dot_general(onehot, cat, (((1,), (0,)), ((), ())),
                         preferred_element_type=jnp.float32)

    wsel_ref[0] = jnp.where(j_col < cc, tw[:, 1:2], 0.0)
    tsel_ref[0] = tw[:, 0:1].reshape(1, _KP).astype(jnp.int32)


def _gather_body(tsel_sref, wsel_ref, attn_ref, out_ref, buf, sem):
    B = out_ref.shape[0] if False else None
    b = pl.program_id(0)
    nb = pl.num_programs(0)

    def issue(bb, par):
        def one(j, carry):
            t = tsel_sref[bb, j]
            pltpu.make_async_copy(
                attn_ref.at[_LAYER, bb, :, pl.ds(t, 1), :],
                buf.at[par, j],
                sem.at[par],
            ).start()
            return carry
        lax.fori_loop(0, _KP, one, 0)

    @pl.when(b == 0)
    def _():
        issue(0, 0)

    @pl.when(b + 1 < nb)
    def _():
        issue(b + 1, (b + 1) % 2)

    par = b % 2

    def wait_one(j, carry):
        t = tsel_sref[b, j]
        pltpu.make_async_copy(
            attn_ref.at[_LAYER, b, :, pl.ds(t, 1), :],
            buf.at[par, j],
            sem.at[par],
        ).wait()
        return carry

    lax.fori_loop(0, _KP, wait_one, 0)

    g = buf[par]
    acc = jnp.sum(g, axis=(1, 2))
    vals = jnp.maximum(wsel_ref[0] * (acc * (1.0 / _H)), 0.0)
    tot = jnp.max(vals, axis=0, keepdims=True)
    shifted = tot - jnp.min(tot)
    div = jnp.clip(jnp.max(shifted), 1e-12, 1.0)
    out_ref[0, 0] = shifted[0] / div


def kernel(fore_map, fore_rep_encoded, target_embed, align_attns, targets):
    B, T, D = target_embed.shape
    M = align_attns.shape[-1]

    fr3 = fore_rep_encoded.reshape(B, 1, D)
    tgt3 = targets.reshape(B, 1, T)

    tsel, wsel = pl.pallas_call(
        _wsel_body,
        grid=(B,),
        in_specs=[
            pl.BlockSpec((1, T, D), lambda b: (b, 0, 0)),
            pl.BlockSpec((1, 1, D), lambda b: (b, 0, 0)),
            pl.BlockSpec((1, 1, T), lambda b: (b, 0, 0)),
        ],
        out_specs=[
            pl.BlockSpec((1, 1, _KP), lambda b: (b, 0, 0)),
            pl.BlockSpec((1, _KP, 1), lambda b: (b, 0, 0)),
        ],
        out_shape=[
            jax.ShapeDtypeStruct((B, 1, _KP), jnp.int32),
            jax.ShapeDtypeStruct((B, _KP, 1), jnp.float32),
        ],
    )(target_embed, fr3, tgt3)

    grid_spec = pltpu.PrefetchScalarGridSpec(
        num_scalar_prefetch=1,
        grid=(B,),
        in_specs=[
            pl.BlockSpec((1, _KP, 1), lambda b, ts: (b, 0, 0)),
            pl.BlockSpec(memory_space=pl.ANY),
        ],
        out_specs=pl.BlockSpec((1, 1, M), lambda b, ts: (b, 0, 0)),
        scratch_shapes=[
            pltpu.VMEM((2, _KP, _H, 1, M), jnp.float32),
            pltpu.SemaphoreType.DMA((2,)),
        ],
    )

    total_attn = pl.pallas_call(
        _gather_body,
        grid_spec=grid_spec,
        out_shape=jax.ShapeDtypeStruct((B, 1, M), jnp.float32),
    )(tsel.reshape(B, _KP), wsel, align_attns)

    return (jnp.squeeze(fore_map, axis=1), total_attn.reshape(B, M))


# speedup vs baseline: 1.0856x; 1.0856x over previous
import jax
import jax.numpy as jnp
from jax import lax
from jax.experimental import pallas as pl
from jax.experimental.pallas import tpu as pltpu

_TOPK = 0.1
_LAYER = 2
_K = 51
_H = 8


def _fused_body(te_ref, fr_ref, tgt_ref, attn_ref, out_ref):
    T, D = te_ref.shape[1], te_ref.shape[2]
    te = te_ref[0]
    fr = fr_ref[0]
    tgt = tgt_ref[0]

    dn = (((1,), (1,)), ((), ()))
    num_row = lax.dot_general(fr, te, dn,
                              preferred_element_type=jnp.float32)
    ones_d = jnp.ones((1, D), jnp.float32)
    sqn_row = lax.dot_general(ones_d, te * te, dn,
                              preferred_element_type=jnp.float32)
    fr_norm = jnp.sqrt(jnp.sum(fr * fr))
    w_row = num_row / jnp.maximum(jnp.sqrt(sqn_row) * fr_norm, 1e-8)
    w_col = w_row.reshape(T, 1)

    ic = lax.broadcasted_iota(jnp.int32, (T, T), 0)
    ir = lax.broadcasted_iota(jnp.int32, (T, T), 1)
    beats = (w_col > w_row) | ((w_col == w_row) & (ic < ir))
    beats_f = jnp.where(beats, 1.0, 0.0)
    ones_t = jnp.ones((1, T), jnp.float32)
    rank_row = lax.dot_general(ones_t, beats_f, (((1,), (0,)), ((), ())),
                               preferred_element_type=jnp.float32)

    seq_len = jnp.sum((tgt != 0).astype(jnp.float32))
    cc = jnp.minimum(jnp.ceil(seq_len * _TOPK), float(_K))

    wm_col = jnp.where(rank_row < cc, w_row, 0.0).reshape(T, 1)

    acc = attn_ref[0, 0, 0]
    for h in range(1, _H):
        acc = acc + attn_ref[0, 0, h]
    vals = jnp.maximum(wm_col * (acc * (1.0 / _H)), 0.0)
    tot = jnp.max(vals, axis=0, keepdims=True)

    shifted = tot - jnp.min(tot)
    div = jnp.clip(jnp.max(shifted), 1e-12, 1.0)
    out_ref[0, 0] = shifted[0] / div


def kernel(fore_map, fore_rep_encoded, target_embed, align_attns, targets):
    B, T, D = target_embed.shape
    M = align_attns.shape[-1]

    fr3 = fore_rep_encoded.reshape(B, 1, D)
    tgt3 = targets.reshape(B, 1, T)

    total_attn = pl.pallas_call(
        _fused_body,
        grid=(B,),
        in_specs=[
            pl.BlockSpec((1, T, D), lambda b: (b, 0, 0)),
            pl.BlockSpec((1, 1, D), lambda b: (b, 0, 0)),
            pl.BlockSpec((1, 1, T), lambda b: (b, 0, 0)),
            pl.BlockSpec((1, 1, _H, T, M), lambda b: (_LAYER, b, 0, 0, 0)),
        ],
        out_specs=pl.BlockSpec((1, 1, M), lambda b: (b, 0, 0)),
        out_shape=jax.ShapeDtypeStruct((B, 1, M), jnp.float32),
    )(target_embed, fr3, tgt3, align_attns)

    return (jnp.squeeze(fore_map, axis=1), total_attn.reshape(B, M))
